# baseline (speedup 1.0000x reference)
import functools

import jax
import jax.numpy as jnp
from jax import lax
from jax.experimental import pallas as pl
from jax.experimental.pallas import tpu as pltpu
from jax.experimental.pallas import tpu_sc as plsc

VOCAB = 1000000
EMBED = 64
BATCH = 16384
HIST = 50

NC = 2
NS = 16
NW = NC * NS
L = 16

BB = 128
PP = 129
NBLK = HIST * (BATCH // BB)
BPW = NBLK // NW

_mesh = plsc.VectorSubcoreMesh(core_axis_name="c", subcore_axis_name="s")


@functools.partial(
    pl.kernel,
    mesh=_mesh,
    out_type=jax.ShapeDtypeStruct((HIST, EMBED, BATCH), jnp.float32),
    compiler_params=pltpu.CompilerParams(needs_layout_passes=False),
    scratch_types=[
        pltpu.VMEM((2, BB), jnp.int32),
        pltpu.VMEM((2, BB), jnp.int32),
        pltpu.VMEM((2, BB), jnp.int32),
        pltpu.VMEM((2, BB, PP), jnp.float32),
        pltpu.VMEM((2, EMBED, BB), jnp.float32),
        pltpu.SemaphoreType.DMA((2,)),
        pltpu.SemaphoreType.DMA((2,)),
        pltpu.SemaphoreType.DMA((2,)),
    ],
)
def _emb_lookup(x_hbm, a_hbm, out_hbm, idx_v, row_v, cb_v, pair_v, tile_v,
                isem, gsem, wsem):
    wid = lax.axis_index("s") * NC + lax.axis_index("c")
    blk0 = wid * BPW

    jiota = [lax.iota(jnp.int32, L) + jb * L for jb in range(BB // L)]

    def start_idx(t, b):
        blk = blk0 + t
        pltpu.async_copy(
            x_hbm.at[blk >> 7, pl.ds((blk & 127) * BB, BB)], idx_v.at[b],
            isem.at[b])

    def wait_idx(b):
        pltpu.make_async_copy(
            x_hbm.at[0, pl.ds(0, BB)], idx_v.at[b], isem.at[b]).wait()

    def gather_dst(b):
        return pair_v.at[b, slice(None), pl.ds(0, BB)]

    def wait_out(b):
        pltpu.make_async_copy(
            tile_v.at[b], out_hbm.at[0, pl.ds(0, EMBED), pl.ds(0, BB)],
            wsem.at[b]).wait()

    start_idx(0, 0)

    def step(t, b):
        bo = 1 - b

        @pl.when(t < BPW)
        def _():
            wait_idx(b)
            for jb in range(BB // L):
                v = idx_v[b, pl.ds(jb * L, L)]
                row_v[b, pl.ds(jb * L, L)] = v >> 1
                cb_v[b, pl.ds(jb * L, L)] = (v & 1) << 6
            pltpu.async_copy(a_hbm.at[row_v.at[b]], gather_dst(b), gsem.at[b])

        @pl.when(t + 1 < BPW)
        def _():
            start_idx(t + 1, bo)

        @pl.when((t >= 1) & (t <= BPW))
        def _():
            @pl.when(t >= 3)
            def _():
                wait_out(bo)
            colbase = [cb_v[bo, pl.ds(jb * L, L)] for jb in range(BB // L)]
            pltpu.make_async_copy(
                a_hbm.at[row_v.at[bo]], gather_dst(bo), gsem.at[bo]).wait()

            def erow(e, c):
                for jb in range(BB // L):
                    vals = plsc.load_gather(
                        pair_v.at[bo], [jiota[jb], colbase[jb] + e])
                    tile_v[bo, e, pl.ds(jb * L, L)] = vals
                return c
            lax.fori_loop(0, EMBED, erow, 0, unroll=8)

            blk = blk0 + t - 1
            pltpu.async_copy(
                tile_v.at[bo],
                out_hbm.at[blk >> 7, pl.ds(0, EMBED),
                           pl.ds((blk & 127) * BB, BB)],
                wsem.at[bo])

    def pair(k, carry):
        step(2 * k, 0)
        step(2 * k + 1, 1)
        return carry

    lax.fori_loop(0, BPW // 2 + 1, pair, 0)

    wait_out(0)
    wait_out(1)


def kernel(x, A):
    out = _emb_lookup(x.T, A.reshape(VOCAB // 2, 128))
    return jnp.transpose(out, (2, 0, 1))


# speedup vs baseline: 1.4761x; 1.4761x over previous
import functools

import jax
import jax.numpy as jnp
from jax import lax
from jax.experimental import pallas as pl
from jax.experimental.pallas import tpu as pltpu
from jax.experimental.pallas import tpu_sc as plsc

VOCAB = 1000000
EMBED = 64
BATCH = 16384
HIST = 50

NC = 2
NS = 16
NW = NC * NS
BPW = BATCH // NW

BBLK = 8
NBLK = BPW // BBLK
NBUF = 2

_mesh = plsc.VectorSubcoreMesh(core_axis_name="c", subcore_axis_name="s")


@functools.partial(
    pl.kernel,
    mesh=_mesh,
    out_type=jax.ShapeDtypeStruct((BATCH, HIST, EMBED), jnp.float32),
    compiler_params=pltpu.CompilerParams(
        use_tc_tiling_on_sc=False, needs_layout_passes=False),
    scratch_types=[
        pltpu.VMEM((NBUF, BBLK, HIST), jnp.int32),
        pltpu.VMEM((NBUF, BBLK, HIST, EMBED), jnp.float32),
        pltpu.SemaphoreType.DMA((NBUF,)),
        pltpu.SemaphoreType.DMA,
        pltpu.SemaphoreType.DMA((NBUF,)),
    ],
)
def _emb_lookup(x_hbm, a_hbm, out_hbm, idx_v, rows_v, isem, gsem, wsem):
    wid = lax.axis_index("s") * NC + lax.axis_index("c")
    row0 = wid * BPW

    def start_idx(i, b):
        pltpu.async_copy(
            x_hbm.at[pl.ds(row0 + i * BBLK, BBLK)], idx_v.at[b], isem.at[b])

    def drain_idx(b):
        pltpu.make_async_copy(
            x_hbm.at[pl.ds(0, BBLK)], idx_v.at[b], isem.at[b]).wait()

    def drain_write(b):
        pltpu.make_async_copy(
            rows_v.at[b], out_hbm.at[pl.ds(0, BBLK)], wsem.at[b]).wait()

    for b in range(NBUF):
        start_idx(b, b)

    def step(i0, carry):
        for b in range(NBUF):
            i = i0 + b
            drain_idx(b)
            @pl.when(i >= NBUF)
            def _():
                drain_write(b)
            copies = [
                pltpu.async_copy(
                    a_hbm.at[idx_v.at[b].at[k]], rows_v.at[b].at[k], gsem)
                for k in range(BBLK)
            ]
            for c in copies:
                c.wait()
            @pl.when(i + NBUF < NBLK)
            def _():
                start_idx(i + NBUF, b)
            pltpu.async_copy(
                rows_v.at[b], out_hbm.at[pl.ds(row0 + i * BBLK, BBLK)],
                wsem.at[b])
        return carry

    lax.fori_loop(0, NBLK // NBUF, lambda k, c: step(k * NBUF, c), 0)

    for b in range(NBUF):
        drain_write(b)


def kernel(x, A):
    return _emb_lookup(x.astype(jnp.int32), A)
